# baseline (speedup 1.0000x reference)
import functools

import jax
import jax.numpy as jnp
from jax import lax
from jax.experimental import pallas as pl
from jax.experimental.pallas import tpu as pltpu
from jax.experimental.pallas import tpu_sc as plsc


@functools.lru_cache(maxsize=None)
def _make_kernel(batch: int, table_len: int):
    info = plsc.get_sparse_core_info()
    nc, ns, lanes = 1, info.num_subcores, info.num_lanes
    nw = nc * ns
    assert batch % (nw * lanes) == 0
    bpw = batch // nw
    mesh = plsc.VectorSubcoreMesh(
        core_axis_name="c", subcore_axis_name="s", num_cores=nc
    )

    @functools.partial(
        pl.kernel,
        mesh=mesh,
        out_type=jax.ShapeDtypeStruct((batch,), jnp.float32),
        compiler_params=pltpu.CompilerParams(needs_layout_passes=False),
        scratch_types=[
            pltpu.VMEM((bpw,), jnp.int32),
            pltpu.VMEM((bpw,), jnp.float32),
            pltpu.SemaphoreType.DMA,
        ],
    )
    def k(t_hbm, betas_hbm, out_hbm, idx_v, out_v, sem_g):
        wid = lax.axis_index("s") * nc + lax.axis_index("c")
        base = wid * bpw
        pltpu.sync_copy(t_hbm.at[pl.ds(base, bpw)], idx_v)
        pltpu.async_copy(betas_hbm.at[idx_v], out_v, sem_g).wait()
        pltpu.sync_copy(out_v, out_hbm.at[pl.ds(base, bpw)])

    return k


def kernel(t_int, betas):
    return _make_kernel(t_int.shape[0], betas.shape[0])(
        t_int.astype(jnp.int32), betas
    )
